# baseline (speedup 1.0000x reference)
import functools

import jax
import jax.numpy as jnp
from jax import lax
from jax.experimental import pallas as pl
from jax.experimental.pallas import tpu as pltpu
from jax.experimental.pallas import tpu_sc as plsc

VOCAB = 100000
D = 64
B = 4096
CTX = 20
NC = 2
NS = 16
NW = NC * NS
BPW = B // NW
LANES = 16


def _sc_pool_body(idx_hbm, table_hbm, out_hbm, idx_v, rows_v, acc_v, sem):
    wid = lax.axis_index("s") * NC + lax.axis_index("c")
    base = wid * BPW
    pltpu.sync_copy(idx_hbm.at[wid], idx_v)
    for j in range(CTX):
        pltpu.async_copy(table_hbm.at[idx_v.at[j]], rows_v, sem).wait()
        if j == 0:
            def body(i, carry):
                for c in range(D // LANES):
                    sl = pl.ds(c * LANES, LANES)
                    acc_v[i, sl] = rows_v[i, sl]
                return carry
        elif j == CTX - 1:
            def body(i, carry):
                for c in range(D // LANES):
                    sl = pl.ds(c * LANES, LANES)
                    acc_v[i, sl] = (acc_v[i, sl] + rows_v[i, sl]) * (1.0 / CTX)
                return carry
        else:
            def body(i, carry):
                for c in range(D // LANES):
                    sl = pl.ds(c * LANES, LANES)
                    acc_v[i, sl] = acc_v[i, sl] + rows_v[i, sl]
                return carry
        lax.fori_loop(0, BPW, body, 0)
    pltpu.sync_copy(acc_v, out_hbm.at[pl.ds(base, BPW)])


_sc_pool = pl.kernel(
    _sc_pool_body,
    out_type=jax.ShapeDtypeStruct((B, D), jnp.float32),
    mesh=plsc.VectorSubcoreMesh(core_axis_name="c", subcore_axis_name="s"),
    scratch_types=[
        pltpu.VMEM((CTX, BPW), jnp.int32),
        pltpu.VMEM((BPW, D), jnp.float32),
        pltpu.VMEM((BPW, D), jnp.float32),
        pltpu.SemaphoreType.DMA,
    ],
    compiler_params=pltpu.CompilerParams(use_tc_tiling_on_sc=False),
)

VB = 512


def _mm_body(p_ref, w_ref, o_ref):
    o_ref[...] = lax.dot_general(
        p_ref[...], w_ref[...],
        dimension_numbers=(((1,), (1,)), ((), ())),
        preferred_element_type=jnp.float32,
    )


def _matmul(pooled, w):
    return pl.pallas_call(
        _mm_body,
        grid=(pl.cdiv(VOCAB, VB),),
        in_specs=[
            pl.BlockSpec((B, D), lambda j: (0, 0)),
            pl.BlockSpec((VB, D), lambda j: (j, 0)),
        ],
        out_specs=pl.BlockSpec((B, VB), lambda j: (0, j)),
        out_shape=jax.ShapeDtypeStruct((B, VOCAB), jnp.float32),
    )(pooled, w)


@jax.jit
def kernel(context_words, emb_table, linear_w):
    idx = context_words.astype(jnp.int32).T.reshape(CTX, NW, BPW)
    idx = idx.transpose(1, 0, 2)
    pooled = _sc_pool(idx, emb_table)
    return _matmul(pooled, linear_w)
